# baseline (speedup 1.0000x reference)
import functools

import jax
import jax.numpy as jnp
from jax import lax
from jax.experimental import pallas as pl
from jax.experimental.pallas import tpu as pltpu
from jax.experimental.pallas import tpu_sc as plsc

E = 8
EPAD = 16
T = 2048
D = 768
H = 512
COEF = 0.01
NEG = -1e30
TR = 128
SORT_ROWS = 5120
NTS = SORT_ROWS // TR
NW = 32


def _cumsum_sub_excl(a):
    n = a.shape[0]
    incl = a
    s = 1
    while s < n:
        incl = incl + jnp.pad(incl, ((s, 0), (0, 0)))[:n]
        s *= 2
    return incl - a


def _cumsum_lane_excl(a):
    n = a.shape[1]
    a = jnp.pad(a, ((0, 0), (1, 0)))[:, :n]
    s = 1
    while s < n:
        a = a + jnp.pad(a, ((0, 0), (s, 0)))[:, :n]
        s *= 2
    return a


def _router_body(x_ref, wr_ref, aux_ref, tpose_ref, pb1_ref, pb2_ref, tiles_ref):
    x = x_ref[...]
    wr = wr_ref[...]
    logits = lax.dot_general(x, wr, (((1,), (1,)), ((), ())),
                             preferred_element_type=jnp.float32)
    lane = lax.broadcasted_iota(jnp.int32, (T, EPAD), 1)
    valid = lane < E
    logits = jnp.where(valid, logits, NEG)

    m = jnp.max(logits, axis=1, keepdims=True)
    ex = jnp.exp(logits - m)
    ex = jnp.where(valid, ex, 0.0)
    probs = ex / jnp.sum(ex, axis=1, keepdims=True)

    i1 = jnp.argmax(logits, axis=1).reshape(T, 1)
    oh1 = (lane == i1).astype(jnp.float32)
    p1 = jnp.max(probs, axis=1, keepdims=True)
    logits2 = jnp.where(lane == i1, NEG, logits)
    i2 = jnp.argmax(logits2, axis=1).reshape(T, 1)
    oh2 = (lane == i2).astype(jnp.float32)
    p2 = jnp.max(jnp.where(lane == i1, NEG, probs), axis=1, keepdims=True)

    density = jnp.mean(oh1, axis=0, keepdims=True)
    rpm = jnp.mean(probs, axis=0, keepdims=True)
    aux = COEF * jnp.sum(density * rpm) * E
    aux_ref[...] = jnp.full((8, 128), aux, dtype=jnp.float32)

    c0 = _cumsum_sub_excl(oh1)
    c1 = _cumsum_sub_excl(oh2)
    total0 = jnp.sum(oh1, axis=0, keepdims=True)
    total1 = jnp.sum(oh2, axis=0, keepdims=True)
    counts = total0 + total1
    cnt_al = jnp.floor((counts + (TR - 1)) / TR) * TR
    base = _cumsum_lane_excl(cnt_al)

    rank0 = jnp.sum(c0 * oh1, axis=1, keepdims=True)
    rank1 = jnp.sum((c1 + total0) * oh2, axis=1, keepdims=True)
    base0 = jnp.sum(base * oh1, axis=1, keepdims=True)
    base1 = jnp.sum(base * oh2, axis=1, keepdims=True)
    pos0f = base0 + rank0
    pos1f = base1 + rank1

    lane8 = lax.broadcasted_iota(jnp.int32, (T, 8), 1)
    tpose_ref[...] = pos0f * (lane8 == 0) + pos1f * (lane8 == 1)

    pb1_ref[...] = p1 * jnp.ones((1, 128), jnp.float32)
    pb2_ref[...] = p2 * jnp.ones((1, 128), jnp.float32)

    s_i = lax.broadcasted_iota(jnp.int32, (64, EPAD), 0).astype(jnp.float32) * TR
    lane64 = lax.broadcasted_iota(jnp.int32, (64, EPAD), 1)
    hit = ((s_i >= base) & (lane64 < E)).astype(jnp.float32)
    te_col = jnp.sum(hit, axis=1, keepdims=True) - 1.0
    total_al = jnp.sum(jnp.where(lane64[:1] < E, cnt_al, 0.0),
                       axis=1, keepdims=True)
    tv_col = (s_i[:, 0:1] < total_al).astype(jnp.float32)
    lane8b = lax.broadcasted_iota(jnp.int32, (64, 8), 1)
    tiles_ref[...] = te_col * (lane8b == 0) + tv_col * (lane8b == 1)


def _mlp(x, w1, w3, w2):
    h1 = lax.dot_general(x, w1, (((1,), (1,)), ((), ())),
                         preferred_element_type=jnp.float32)
    h3 = lax.dot_general(x, w3, (((1,), (1,)), ((), ())),
                         preferred_element_type=jnp.float32)
    g = h1 * jax.nn.sigmoid(h1) * h3
    return lax.dot_general(g, w2, (((1,), (1,)), ((), ())),
                           preferred_element_type=jnp.float32)


def _shared_body(x_ref, w1_ref, w3_ref, w2_ref, out_ref):
    out_ref[...] = _mlp(x_ref[...], w1_ref[0], w3_ref[0], w2_ref[0])


def _routed_body(te_ref, tv_ref, xs_ref, w1_ref, w3_ref, w2_ref, out_ref):
    i = pl.program_id(0)

    @pl.when(tv_ref[i] == 1)
    def _():
        out_ref[...] = _mlp(xs_ref[...], w1_ref[0], w3_ref[0], w2_ref[0])


def _scatter_body(x_hbm, dst_hbm, xs_hbm, idx_v, rows_v, sem):
    wid = lax.axis_index("s") * 2 + lax.axis_index("c")
    for b in range(2):
        c = wid * 2 + b
        base_src = lax.rem(c, 32) * 64
        pltpu.sync_copy(dst_hbm.at[pl.ds(c * 64, 64)], idx_v)
        pltpu.sync_copy(x_hbm.at[pl.ds(base_src, 64)], rows_v)
        pltpu.async_copy(rows_v, xs_hbm.at[idx_v], sem).wait()


def _combine_body(bufr_hbm, bufsh_hbm, pos0_hbm, pos1_hbm, pb1_hbm, pb2_hbm,
                  out_hbm, idx0_v, idx1_v, r0_v, r1_v, acc_v, pb1_v, pb2_v,
                  sem0, sem1):
    wid = lax.axis_index("s") * 2 + lax.axis_index("c")
    for b in range(2):
        tok0 = wid * 64 + b * 32
        pltpu.sync_copy(pos0_hbm.at[pl.ds(tok0, 32)], idx0_v)
        pltpu.sync_copy(pos1_hbm.at[pl.ds(tok0, 32)], idx1_v)
        cp0 = pltpu.async_copy(bufr_hbm.at[idx0_v], r0_v, sem0)
        cp1 = pltpu.async_copy(bufr_hbm.at[idx1_v], r1_v, sem1)
        pltpu.sync_copy(bufsh_hbm.at[pl.ds(tok0, 32)], acc_v)
        pltpu.sync_copy(pb1_hbm.at[pl.ds(tok0, 32)], pb1_v)
        pltpu.sync_copy(pb2_hbm.at[pl.ds(tok0, 32)], pb2_v)
        cp0.wait()
        cp1.wait()

        def body(t, carry):
            a1 = pb1_v[t, pl.ds(0, 16)]
            a2 = pb2_v[t, pl.ds(0, 16)]
            for ch in range(48):
                sl = pl.ds(ch * 16, 16)
                acc_v[t, sl] = acc_v[t, sl] + a1 * r0_v[t, sl] + a2 * r1_v[t, sl]
            return carry

        lax.fori_loop(0, 32, body, 0)
        pltpu.sync_copy(acc_v, out_hbm.at[pl.ds(tok0, 32)])


@functools.lru_cache(maxsize=1)
def _sc_kernels():
    mesh = plsc.VectorSubcoreMesh(core_axis_name="c", subcore_axis_name="s")
    scatter_k = pl.kernel(
        _scatter_body,
        out_type=jax.ShapeDtypeStruct((SORT_ROWS, D), jnp.float32),
        mesh=mesh,
        scratch_types=[
            pltpu.VMEM((64,), jnp.int32),
            pltpu.VMEM((64, D), jnp.float32),
            pltpu.SemaphoreType.DMA,
        ],
    )
    combine_k = pl.kernel(
        _combine_body,
        out_type=jax.ShapeDtypeStruct((T, D), jnp.float32),
        mesh=mesh,
        scratch_types=[
            pltpu.VMEM((32,), jnp.int32),
            pltpu.VMEM((32,), jnp.int32),
            pltpu.VMEM((32, D), jnp.float32),
            pltpu.VMEM((32, D), jnp.float32),
            pltpu.VMEM((32, D), jnp.float32),
            pltpu.VMEM((32, 128), jnp.float32),
            pltpu.VMEM((32, 128), jnp.float32),
            pltpu.SemaphoreType.DMA,
            pltpu.SemaphoreType.DMA,
        ],
    )
    return scatter_k, combine_k


def kernel(x, Wr, W1, W2, W3, sW1, sW2, sW3):
    B, S, Dm = x.shape
    x_flat = x.reshape(T, D)
    wr_pad = jnp.zeros((EPAD, D), jnp.float32).at[:E].set(Wr)

    aux, tpose, pb1, pb2, tiles8 = pl.pallas_call(
        _router_body,
        out_shape=(
            jax.ShapeDtypeStruct((8, 128), jnp.float32),
            jax.ShapeDtypeStruct((T, 8), jnp.float32),
            jax.ShapeDtypeStruct((T, 128), jnp.float32),
            jax.ShapeDtypeStruct((T, 128), jnp.float32),
            jax.ShapeDtypeStruct((64, 8), jnp.float32),
        ),
    )(x_flat, wr_pad)

    pos0 = tpose[:, 0].astype(jnp.int32)
    pos1 = tpose[:, 1].astype(jnp.int32)
    dst = jnp.concatenate([pos0, pos1])
    te = tiles8[:NTS, 0].astype(jnp.int32)
    tv = tiles8[:NTS, 1].astype(jnp.int32)

    scatter_k, combine_k = _sc_kernels()
    x_sorted = scatter_k(x_flat, dst)

    bufsh = pl.pallas_call(
        _shared_body,
        in_specs=[
            pl.BlockSpec((T, D), lambda: (0, 0)),
            pl.BlockSpec((1, H, D), lambda: (0, 0, 0)),
            pl.BlockSpec((1, H, D), lambda: (0, 0, 0)),
            pl.BlockSpec((1, D, H), lambda: (0, 0, 0)),
        ],
        out_specs=pl.BlockSpec((T, D), lambda: (0, 0)),
        out_shape=jax.ShapeDtypeStruct((T, D), jnp.float32),
    )(x_flat, sW1, sW3, sW2)

    bufr = pl.pallas_call(
        _routed_body,
        grid_spec=pltpu.PrefetchScalarGridSpec(
            num_scalar_prefetch=2,
            grid=(NTS,),
            in_specs=[
                pl.BlockSpec((TR, D), lambda i, te_r, tv_r: (i, 0)),
                pl.BlockSpec((1, H, D), lambda i, te_r, tv_r: (te_r[i], 0, 0)),
                pl.BlockSpec((1, H, D), lambda i, te_r, tv_r: (te_r[i], 0, 0)),
                pl.BlockSpec((1, D, H), lambda i, te_r, tv_r: (te_r[i], 0, 0)),
            ],
            out_specs=pl.BlockSpec((TR, D), lambda i, te_r, tv_r: (i, 0)),
        ),
        out_shape=jax.ShapeDtypeStruct((SORT_ROWS, D), jnp.float32),
    )(te, tv, x_sorted, W1, W3, W2)

    out = combine_k(bufr, bufsh, pos0, pos1, pb1, pb2)

    return out.reshape(B, S, Dm), aux[0, 0]
